# baseline (speedup 1.0000x reference)
import jax
import jax.numpy as jnp
from jax import lax
from jax.experimental import pallas as pl
from jax.experimental.pallas import tpu as pltpu
from jax.experimental.pallas import tpu_sc as plsc

EMBED_DIM = 64
LANES = 16
NUM_CORES = 2
NUM_SUBCORES = 16
NUM_WORKERS = NUM_CORES * NUM_SUBCORES
SLICES = EMBED_DIM // LANES
VOCAB = 53


def _make_body(B, G):
    rows_per_batch = G
    batch_per_worker = B // NUM_WORKERS

    def _body(idx_hbm, x_hbm, tab_hbm, w_hbm, b_hbm, out_hbm,
              tab_v, w_v, b_v, idx_v, x_v,
              rows_a, out_sem_a, rows_b, out_sem_b):
        wid = lax.axis_index("s") * NUM_CORES + lax.axis_index("c")
        batch0 = pl.multiple_of(wid * batch_per_worker, 8)
        worker_base = wid * batch_per_worker * rows_per_batch

        pltpu.sync_copy(w_hbm, w_v)
        pltpu.sync_copy(b_hbm, b_v)
        pltpu.sync_copy(tab_hbm, tab_v)
        pltpu.sync_copy(idx_hbm.at[pl.ds(batch0, batch_per_worker)], idx_v)
        pltpu.sync_copy(x_hbm.at[pl.ds(batch0, batch_per_worker)], x_v)
        w_regs = [w_v[pl.ds(c * LANES, LANES)] for c in range(SLICES)]
        b_regs = [b_v[pl.ds(c * LANES, LANES)] for c in range(SLICES)]

        def fold_row(v, _):
            for c in range(SLICES):
                sl = pl.ds(c * LANES, LANES)
                tab_v[v, sl] = tab_v[v, sl] + b_regs[c]
            return _

        lax.fori_loop(0, VOCAB, fold_row, None)

        col_regs = [c * LANES + lax.iota(jnp.int32, LANES)
                    for c in range(SLICES)]

        def store_wait(bb, rows_v, sem):
            base = worker_base + bb * rows_per_batch
            pltpu.make_async_copy(
                rows_v, out_hbm.at[pl.ds(base, rows_per_batch)], sem).wait()

        def chunk(bb, rows_v, sem):
            @pl.when(bb >= 2)
            def _drain():
                store_wait(bb - 2, rows_v, sem)

            lane_bb = jnp.broadcast_to(bb, (LANES,))

            @plsc.parallel_loop(0, rows_per_batch, step=1, unroll=8)
            def row_body(g):
                lane_g = jnp.broadcast_to(g, (LANES,))
                iv = plsc.load_gather(idx_v, [lane_bb, lane_g])
                xs = plsc.load_gather(x_v, [lane_bb, lane_g])
                for c in range(SLICES):
                    tr = plsc.load_gather(tab_v, [iv, col_regs[c]])
                    rows_v[g, pl.ds(c * LANES, LANES)] = tr + xs * w_regs[c]

            base = worker_base + bb * rows_per_batch
            pltpu.async_copy(rows_v, out_hbm.at[pl.ds(base, rows_per_batch)],
                             sem)

        def pair_body(bp, _):
            chunk(bp * 2, rows_a, out_sem_a)
            chunk(bp * 2 + 1, rows_b, out_sem_b)
            return _

        lax.fori_loop(0, batch_per_worker // 2, pair_body, None)
        store_wait(batch_per_worker - 2, rows_a, out_sem_a)
        store_wait(batch_per_worker - 1, rows_b, out_sem_b)

    return _body


def kernel(discrete_expression, normalized_expr, bin_table, W, b):
    B, G = discrete_expression.shape
    N = B * G
    idx = discrete_expression.astype(jnp.int32)
    w = W[:, 0]

    mesh = plsc.VectorSubcoreMesh(core_axis_name="c", subcore_axis_name="s")
    run = pl.kernel(
        _make_body(B, G),
        out_type=jax.ShapeDtypeStruct((N, EMBED_DIM), jnp.float32),
        mesh=mesh,
        compiler_params=pltpu.CompilerParams(needs_layout_passes=False),
        scratch_types=[
            pltpu.VMEM((VOCAB, EMBED_DIM), jnp.float32),
            pltpu.VMEM((EMBED_DIM,), jnp.float32),
            pltpu.VMEM((EMBED_DIM,), jnp.float32),
            pltpu.VMEM((B // NUM_WORKERS, G), jnp.int32),
            pltpu.VMEM((B // NUM_WORKERS, G), jnp.float32),
            pltpu.VMEM((G, EMBED_DIM), jnp.float32),
            pltpu.SemaphoreType.DMA,
            pltpu.VMEM((G, EMBED_DIM), jnp.float32),
            pltpu.SemaphoreType.DMA,
        ],
    )
    out = run(idx, normalized_expr, bin_table, w, b)
    return out.reshape(B, G, EMBED_DIM)
